# baseline (speedup 1.0000x reference)
import functools

import jax
import jax.numpy as jnp
from jax import lax
from jax.experimental import pallas as pl
from jax.experimental.pallas import tpu as pltpu
from jax.experimental.pallas import tpu_sc as plsc

N = 10000
E = 320000
H = 128

NC = 2
NS = 16
NW = NC * NS
EPW = E // NW
C = 40
NPH = 3
NSL = 4
NCHUNK = EPW // C
RPT = N // NS
ZR = 80

_f32 = jnp.float32


def _t1_body(x_ref, encw_ref, encb_ref, wd_ref, ws_ref, b1_ref,
             h_ref, a_ref, b_ref):
    h = jnp.dot(x_ref[...], encw_ref[...],
                preferred_element_type=_f32) + encb_ref[...]
    h_ref[...] = h
    a_ref[...] = jnp.dot(h, wd_ref[...],
                         preferred_element_type=_f32) + b1_ref[...]
    b_ref[...] = jnp.dot(h, ws_ref[...],
                         preferred_element_type=_f32)


def _t1(x, enc_w, enc_b, wd, ws, b1):
    blk = 1000
    grid = N // blk
    return pl.pallas_call(
        _t1_body,
        grid=(grid,),
        in_specs=[
            pl.BlockSpec((blk, 5), lambda i: (i, 0)),
            pl.BlockSpec((5, H), lambda i: (0, 0)),
            pl.BlockSpec((1, H), lambda i: (0, 0)),
            pl.BlockSpec((H, H), lambda i: (0, 0)),
            pl.BlockSpec((H, H), lambda i: (0, 0)),
            pl.BlockSpec((1, H), lambda i: (0, 0)),
        ],
        out_specs=[
            pl.BlockSpec((blk, H), lambda i: (i, 0)),
            pl.BlockSpec((blk, H), lambda i: (i, 0)),
            pl.BlockSpec((blk, H), lambda i: (i, 0)),
        ],
        out_shape=[jax.ShapeDtypeStruct((N, H), _f32)] * 3,
    )(x, enc_w, enc_b, wd, ws, b1)


def _sc_body(a_h, b_h, idx_h, ea_h, we_h,
             z128_h, z16_h, o16_h,
             s_out, deg_out,
             idxq, eaq, wev, arq, brq, onesv,
             s_sh, deg_sh, sem_i, sem_g, sem_s):
    cid = lax.axis_index("c")
    sid = lax.axis_index("s")
    wid = sid * NC + cid
    r0 = sid * RPT

    nfull, rem = RPT // ZR, RPT % ZR
    for q in range(nfull):
        pltpu.sync_copy(z128_h, s_sh.at[pl.ds(r0 + q * ZR, ZR)])
        pltpu.sync_copy(z16_h, deg_sh.at[pl.ds(r0 + q * ZR, ZR)])
    if rem:
        pltpu.sync_copy(z128_h.at[pl.ds(0, rem)],
                        s_sh.at[pl.ds(r0 + nfull * ZR, rem)])
        pltpu.sync_copy(z16_h.at[pl.ds(0, rem)],
                        deg_sh.at[pl.ds(r0 + nfull * ZR, rem)])

    pltpu.sync_copy(we_h, wev)
    pltpu.sync_copy(o16_h, onesv)
    plsc.subcore_barrier()

    wvecs = [[wev[r, pl.ds(k * 16, 16)] for k in range(8)] for r in range(3)]

    def idx_start(j, s):
        row = wid * NCHUNK + j
        pltpu.async_copy(idx_h.at[1, row], idxq.at[s, 0], sem_i.at[s])
        pltpu.async_copy(idx_h.at[0, row], idxq.at[s, 1], sem_i.at[s])
        pltpu.async_copy(ea_h.at[pl.ds(row, 1)], eaq.at[pl.ds(s, 1)],
                         sem_i.at[s])

    def idx_wait(j, s):
        row = wid * NCHUNK + j
        pltpu.make_async_copy(idx_h.at[1, row], idxq.at[s, 0],
                              sem_i.at[s]).wait()
        pltpu.make_async_copy(idx_h.at[0, row], idxq.at[s, 1],
                              sem_i.at[s]).wait()
        pltpu.make_async_copy(ea_h.at[pl.ds(row, 1)],
                              eaq.at[pl.ds(s, 1)], sem_i.at[s]).wait()

    def gather_start(s, p):
        pltpu.async_copy(a_h.at[idxq.at[s, 0]], arq.at[p], sem_g.at[p])
        pltpu.async_copy(b_h.at[idxq.at[s, 1]], brq.at[p], sem_g.at[p])

    def gather_wait(s, p):
        pltpu.make_async_copy(a_h.at[idxq.at[s, 0]], arq.at[p],
                              sem_g.at[p]).wait()
        pltpu.make_async_copy(b_h.at[idxq.at[s, 1]], brq.at[p],
                              sem_g.at[p]).wait()

    def scatter_start(s, p):
        pltpu.async_copy(arq.at[p], s_sh.at[idxq.at[s, 0]], sem_s.at[p],
                         add=True)
        pltpu.async_copy(onesv, deg_sh.at[idxq.at[s, 0]], sem_s.at[p],
                         add=True)

    def scatter_wait(s, p):
        pltpu.make_async_copy(arq.at[p], s_sh.at[idxq.at[s, 0]],
                              sem_s.at[p]).wait()
        pltpu.make_async_copy(onesv, deg_sh.at[idxq.at[s, 0]],
                              sem_s.at[p]).wait()

    def compute(s, p):
        @plsc.parallel_loop(0, C, unroll=4)
        def edge(e):
            gi = jnp.full((16,), e, jnp.int32)
            si = jnp.full((16,), s, jnp.int32)
            c0 = plsc.load_gather(eaq, [si, gi, jnp.full((16,), 0, jnp.int32)])
            c1 = plsc.load_gather(eaq, [si, gi, jnp.full((16,), 1, jnp.int32)])
            c2 = plsc.load_gather(eaq, [si, gi, jnp.full((16,), 2, jnp.int32)])
            for k in range(8):
                sl = pl.ds(k * 16, 16)
                z = arq[p, e, sl] + brq[p, e, sl]
                z = z + c0 * wvecs[0][k] + c1 * wvecs[1][k] + c2 * wvecs[2][k]
                arq[p, e, sl] = jnp.maximum(z, 0.0)

    def chunk(j, carry):
        p = lax.rem(j, NPH)
        s = lax.rem(j, NSL)

        @pl.when(j >= 2)
        def _():
            scatter_wait(lax.rem(j - 2, NSL), lax.rem(j - 2, NPH))

        @pl.when(j + 2 < NCHUNK)
        def _():
            idx_start(j + 2, lax.rem(j + 2, NSL))

        @pl.when(j + 1 < NCHUNK)
        def _():
            s1 = lax.rem(j + 1, NSL)
            idx_wait(j + 1, s1)
            gather_start(s1, lax.rem(j + 1, NPH))

        gather_wait(s, p)
        compute(s, p)
        scatter_start(s, p)
        return carry

    idx_start(0, 0)
    idx_start(1, 1)
    idx_wait(0, 0)
    gather_start(0, 0)
    lax.fori_loop(0, NCHUNK, chunk, 0)
    scatter_wait((NCHUNK - 2) % NSL, (NCHUNK - 2) % NPH)
    scatter_wait((NCHUNK - 1) % NSL, (NCHUNK - 1) % NPH)
    plsc.subcore_barrier()

    pltpu.sync_copy(s_sh.at[pl.ds(r0, RPT)],
                    s_out.at[cid, pl.ds(r0, RPT)])
    pltpu.sync_copy(deg_sh.at[pl.ds(r0, RPT)],
                    deg_out.at[cid, pl.ds(r0, RPT)])


def _sc_call(a, b, idx3d, ea3d, we, z128, z16, o16):
    mesh = plsc.VectorSubcoreMesh(core_axis_name="c", subcore_axis_name="s")
    return pl.kernel(
        _sc_body,
        out_type=[
            jax.ShapeDtypeStruct((NC, N, H), _f32),
            jax.ShapeDtypeStruct((NC, N, 16), _f32),
        ],
        mesh=mesh,
        compiler_params=pltpu.CompilerParams(use_tc_tiling_on_sc=False,
                                             needs_layout_passes=False),
        scratch_types=[
            pltpu.VMEM((NSL, 2, C), jnp.int32),
            pltpu.VMEM((NSL, C, 3), _f32),
            pltpu.VMEM((3, H), _f32),
            pltpu.VMEM((NPH, C, H), _f32),
            pltpu.VMEM((NPH, C, H), _f32),
            pltpu.VMEM((C, 16), _f32),
            pltpu.VMEM_SHARED((N, H), _f32),
            pltpu.VMEM_SHARED((N, 16), _f32),
            pltpu.SemaphoreType.DMA((NSL,)),
            pltpu.SemaphoreType.DMA((NPH,)),
            pltpu.SemaphoreType.DMA((NPH,)),
        ],
    )(a, b, idx3d, ea3d, we, z128, z16, o16)


def _t2_body(h_ref, sp_ref, dp_ref, w2_ref, b2_ref,
             u1h_ref, u1a_ref, ub1_ref, u2_ref, ub2_ref, dw_ref, db_ref,
             o_ref):
    h = h_ref[...]
    s = sp_ref[0] + sp_ref[1]
    deg = dp_ref[0, :, :1] + dp_ref[1, :, :1]
    aggr = jnp.dot(s, w2_ref[...],
                   preferred_element_type=_f32) + deg * b2_ref[...]
    u = jnp.dot(h, u1h_ref[...], preferred_element_type=_f32)
    u = u + jnp.dot(aggr, u1a_ref[...],
                    preferred_element_type=_f32) + ub1_ref[...]
    u = jnp.maximum(u, 0.0)
    hu = jnp.dot(u, u2_ref[...], preferred_element_type=_f32) + ub2_ref[...]
    o_ref[...] = jnp.dot(h + hu, dw_ref[...],
                         preferred_element_type=_f32) + db_ref[...]


def _t2(h, s_part, deg_part, w2, b2, u1h, u1a, ub1, u2, ub2, dw, db):
    blk = 1000
    grid = N // blk
    full = lambda r, c: pl.BlockSpec((r, c), lambda i: (0, 0))
    row = lambda c: pl.BlockSpec((blk, c), lambda i: (i, 0))
    return pl.pallas_call(
        _t2_body,
        grid=(grid,),
        in_specs=[
            row(H),
            pl.BlockSpec((2, blk, H), lambda i: (0, i, 0)),
            pl.BlockSpec((2, blk, 16), lambda i: (0, i, 0)),
            full(H, H), full(1, H),
            full(H, H), full(H, H), full(1, H),
            full(H, H), full(1, H),
            full(H, 5), full(1, 5),
        ],
        out_specs=pl.BlockSpec((blk, 5), lambda i: (i, 0)),
        out_shape=jax.ShapeDtypeStruct((N, 5), _f32),
    )(h, s_part, deg_part, w2, b2, u1h, u1a, ub1, u2, ub2, dw, db)


def kernel(x, edge_index, edge_attr, enc_w, enc_b, dec_w, dec_b,
           msg_w1, msg_b1, msg_w2, msg_b2, upd_w1, upd_b1, upd_w2, upd_b2):
    l = msg_w1.shape[0] - 1
    W = msg_w1[l]
    wd, ws, we = W[:H], W[H:2 * H], W[2 * H:]

    h, a, b = _t1(x, enc_w, enc_b.reshape(1, H), wd, ws,
                  msg_b1[l].reshape(1, H))

    idx3d = edge_index.reshape(2, E // C, C)
    ea3d = edge_attr.reshape(E // C, C, 3)
    z128 = jnp.zeros((ZR, H), _f32)
    z16 = jnp.zeros((ZR, 16), _f32)
    o16 = jnp.ones((C, 16), _f32)

    s_part, deg_part = _sc_call(a, b, idx3d, ea3d, we, z128, z16, o16)

    u1 = upd_w1[l]
    return _t2(h, s_part, deg_part,
               msg_w2[l], msg_b2[l].reshape(1, H),
               u1[:H], u1[H:], upd_b1[l].reshape(1, H),
               upd_w2[l], upd_b2[l].reshape(1, H),
               dec_w, dec_b.reshape(1, 5))


# speedup vs baseline: 1.7441x; 1.7441x over previous
import functools

import jax
import jax.numpy as jnp
from jax import lax
from jax.experimental import pallas as pl
from jax.experimental.pallas import tpu as pltpu
from jax.experimental.pallas import tpu_sc as plsc

N = 10000
E = 320000
H = 128

NC = 2
NS = 16
NW = NC * NS
EPW = E // NW
C = 40
NPH = 3
NSL = 4
NCHUNK = EPW // C
RPT = N // NS
ZR = 80

_f32 = jnp.float32


def _t1_body(x_ref, encw_ref, encb_ref, wd_ref, ws_ref, b1_ref,
             h_ref, a_ref, b_ref):
    h = jnp.dot(x_ref[...], encw_ref[...],
                preferred_element_type=_f32) + encb_ref[...]
    h_ref[...] = h
    a_ref[...] = jnp.dot(h, wd_ref[...],
                         preferred_element_type=_f32) + b1_ref[...]
    b_ref[...] = jnp.dot(h, ws_ref[...],
                         preferred_element_type=_f32)


def _t1(x, enc_w, enc_b, wd, ws, b1):
    blk = 1000
    grid = N // blk
    return pl.pallas_call(
        _t1_body,
        grid=(grid,),
        in_specs=[
            pl.BlockSpec((blk, 5), lambda i: (i, 0)),
            pl.BlockSpec((5, H), lambda i: (0, 0)),
            pl.BlockSpec((1, H), lambda i: (0, 0)),
            pl.BlockSpec((H, H), lambda i: (0, 0)),
            pl.BlockSpec((H, H), lambda i: (0, 0)),
            pl.BlockSpec((1, H), lambda i: (0, 0)),
        ],
        out_specs=[
            pl.BlockSpec((blk, H), lambda i: (i, 0)),
            pl.BlockSpec((blk, H), lambda i: (i, 0)),
            pl.BlockSpec((blk, H), lambda i: (i, 0)),
        ],
        out_shape=[jax.ShapeDtypeStruct((N, H), _f32)] * 3,
    )(x, enc_w, enc_b, wd, ws, b1)


def _sc_body(a_h, b_h, dst_h, src_h, ea0_h, ea1_h, ea2_h, we_h,
             z128_h, z16_h, o16_h,
             s_out, deg_out,
             idxq, eaq, wev, arq, brq, onesv,
             s_sh, deg_sh, sem_i, sem_g, sem_s):
    cid = lax.axis_index("c")
    sid = lax.axis_index("s")
    wid = sid * NC + cid
    r0 = sid * RPT

    nfull, rem = RPT // ZR, RPT % ZR
    for q in range(nfull):
        pltpu.sync_copy(z128_h, s_sh.at[pl.ds(r0 + q * ZR, ZR)])
        pltpu.sync_copy(z16_h, deg_sh.at[pl.ds(r0 + q * ZR, ZR)])
    if rem:
        pltpu.sync_copy(z128_h.at[pl.ds(0, rem)],
                        s_sh.at[pl.ds(r0 + nfull * ZR, rem)])
        pltpu.sync_copy(z16_h.at[pl.ds(0, rem)],
                        deg_sh.at[pl.ds(r0 + nfull * ZR, rem)])

    pltpu.sync_copy(we_h, wev)
    pltpu.sync_copy(o16_h, onesv)
    plsc.subcore_barrier()

    wvecs = [[wev[r, pl.ds(k * 16, 16)] for k in range(8)] for r in range(3)]

    def idx_start(j, s):
        base = (wid * NCHUNK + j) * C
        pltpu.async_copy(dst_h.at[pl.ds(base, C)], idxq.at[s, 0], sem_i.at[s])
        pltpu.async_copy(src_h.at[pl.ds(base, C)], idxq.at[s, 1], sem_i.at[s])
        pltpu.async_copy(ea0_h.at[pl.ds(base, C)], eaq.at[s, 0], sem_i.at[s])
        pltpu.async_copy(ea1_h.at[pl.ds(base, C)], eaq.at[s, 1], sem_i.at[s])
        pltpu.async_copy(ea2_h.at[pl.ds(base, C)], eaq.at[s, 2], sem_i.at[s])

    def idx_wait(j, s):
        base = (wid * NCHUNK + j) * C
        pltpu.make_async_copy(dst_h.at[pl.ds(base, C)], idxq.at[s, 0],
                              sem_i.at[s]).wait()
        pltpu.make_async_copy(src_h.at[pl.ds(base, C)], idxq.at[s, 1],
                              sem_i.at[s]).wait()
        pltpu.make_async_copy(ea0_h.at[pl.ds(base, C)], eaq.at[s, 0],
                              sem_i.at[s]).wait()
        pltpu.make_async_copy(ea1_h.at[pl.ds(base, C)], eaq.at[s, 1],
                              sem_i.at[s]).wait()
        pltpu.make_async_copy(ea2_h.at[pl.ds(base, C)], eaq.at[s, 2],
                              sem_i.at[s]).wait()

    def gather_start(s, p):
        pltpu.async_copy(a_h.at[idxq.at[s, 0]], arq.at[p], sem_g.at[p])
        pltpu.async_copy(b_h.at[idxq.at[s, 1]], brq.at[p], sem_g.at[p])

    def gather_wait(s, p):
        pltpu.make_async_copy(a_h.at[idxq.at[s, 0]], arq.at[p],
                              sem_g.at[p]).wait()
        pltpu.make_async_copy(b_h.at[idxq.at[s, 1]], brq.at[p],
                              sem_g.at[p]).wait()

    def scatter_start(s, p):
        pltpu.async_copy(arq.at[p], s_sh.at[idxq.at[s, 0]], sem_s.at[p],
                         add=True)
        pltpu.async_copy(onesv, deg_sh.at[idxq.at[s, 0]], sem_s.at[p],
                         add=True)

    def scatter_wait(s, p):
        pltpu.make_async_copy(arq.at[p], s_sh.at[idxq.at[s, 0]],
                              sem_s.at[p]).wait()
        pltpu.make_async_copy(onesv, deg_sh.at[idxq.at[s, 0]],
                              sem_s.at[p]).wait()

    def compute(s, p):
        @plsc.parallel_loop(0, C, unroll=4)
        def edge(e):
            gi = jnp.full((16,), e, jnp.int32)
            si = jnp.full((16,), s, jnp.int32)
            c0 = plsc.load_gather(eaq, [si, jnp.full((16,), 0, jnp.int32), gi])
            c1 = plsc.load_gather(eaq, [si, jnp.full((16,), 1, jnp.int32), gi])
            c2 = plsc.load_gather(eaq, [si, jnp.full((16,), 2, jnp.int32), gi])
            for k in range(8):
                sl = pl.ds(k * 16, 16)
                z = arq[p, e, sl] + brq[p, e, sl]
                z = z + c0 * wvecs[0][k] + c1 * wvecs[1][k] + c2 * wvecs[2][k]
                arq[p, e, sl] = jnp.maximum(z, 0.0)

    def chunk(j, carry):
        p = lax.rem(j, NPH)
        s = lax.rem(j, NSL)

        @pl.when(j >= 2)
        def _():
            scatter_wait(lax.rem(j - 2, NSL), lax.rem(j - 2, NPH))

        @pl.when(j + 2 < NCHUNK)
        def _():
            idx_start(j + 2, lax.rem(j + 2, NSL))

        @pl.when(j + 1 < NCHUNK)
        def _():
            s1 = lax.rem(j + 1, NSL)
            idx_wait(j + 1, s1)
            gather_start(s1, lax.rem(j + 1, NPH))

        gather_wait(s, p)
        compute(s, p)
        scatter_start(s, p)
        return carry

    idx_start(0, 0)
    idx_start(1, 1)
    idx_wait(0, 0)
    gather_start(0, 0)
    lax.fori_loop(0, NCHUNK, chunk, 0)
    scatter_wait((NCHUNK - 2) % NSL, (NCHUNK - 2) % NPH)
    scatter_wait((NCHUNK - 1) % NSL, (NCHUNK - 1) % NPH)
    plsc.subcore_barrier()

    pltpu.sync_copy(s_sh.at[pl.ds(r0, RPT)],
                    s_out.at[cid, pl.ds(r0, RPT)])
    pltpu.sync_copy(deg_sh.at[pl.ds(r0, RPT)],
                    deg_out.at[cid, pl.ds(r0, RPT)])


def _sc_call(a, b, dst, src, ea0, ea1, ea2, we, z128, z16, o16):
    mesh = plsc.VectorSubcoreMesh(core_axis_name="c", subcore_axis_name="s")
    return pl.kernel(
        _sc_body,
        out_type=[
            jax.ShapeDtypeStruct((NC, N, H), _f32),
            jax.ShapeDtypeStruct((NC, N, 16), _f32),
        ],
        mesh=mesh,
        compiler_params=pltpu.CompilerParams(use_tc_tiling_on_sc=False,
                                             needs_layout_passes=False),
        scratch_types=[
            pltpu.VMEM((NSL, 2, C), jnp.int32),
            pltpu.VMEM((NSL, 3, C), _f32),
            pltpu.VMEM((3, H), _f32),
            pltpu.VMEM((NPH, C, H), _f32),
            pltpu.VMEM((NPH, C, H), _f32),
            pltpu.VMEM((C, 16), _f32),
            pltpu.VMEM_SHARED((N, H), _f32),
            pltpu.VMEM_SHARED((N, 16), _f32),
            pltpu.SemaphoreType.DMA((NSL,)),
            pltpu.SemaphoreType.DMA((NPH,)),
            pltpu.SemaphoreType.DMA((NPH,)),
        ],
    )(a, b, dst, src, ea0, ea1, ea2, we, z128, z16, o16)


def _t2_body(h_ref, sp_ref, dp_ref, w2_ref, b2_ref,
             u1h_ref, u1a_ref, ub1_ref, u2_ref, ub2_ref, dw_ref, db_ref,
             o_ref):
    h = h_ref[...]
    s = sp_ref[0] + sp_ref[1]
    deg = dp_ref[0, :, :1] + dp_ref[1, :, :1]
    aggr = jnp.dot(s, w2_ref[...],
                   preferred_element_type=_f32) + deg * b2_ref[...]
    u = jnp.dot(h, u1h_ref[...], preferred_element_type=_f32)
    u = u + jnp.dot(aggr, u1a_ref[...],
                    preferred_element_type=_f32) + ub1_ref[...]
    u = jnp.maximum(u, 0.0)
    hu = jnp.dot(u, u2_ref[...], preferred_element_type=_f32) + ub2_ref[...]
    o_ref[...] = jnp.dot(h + hu, dw_ref[...],
                         preferred_element_type=_f32) + db_ref[...]


def _t2(h, s_part, deg_part, w2, b2, u1h, u1a, ub1, u2, ub2, dw, db):
    blk = 1000
    grid = N // blk
    full = lambda r, c: pl.BlockSpec((r, c), lambda i: (0, 0))
    row = lambda c: pl.BlockSpec((blk, c), lambda i: (i, 0))
    return pl.pallas_call(
        _t2_body,
        grid=(grid,),
        in_specs=[
            row(H),
            pl.BlockSpec((2, blk, H), lambda i: (0, i, 0)),
            pl.BlockSpec((2, blk, 16), lambda i: (0, i, 0)),
            full(H, H), full(1, H),
            full(H, H), full(H, H), full(1, H),
            full(H, H), full(1, H),
            full(H, 5), full(1, 5),
        ],
        out_specs=pl.BlockSpec((blk, 5), lambda i: (i, 0)),
        out_shape=jax.ShapeDtypeStruct((N, 5), _f32),
    )(h, s_part, deg_part, w2, b2, u1h, u1a, ub1, u2, ub2, dw, db)


def kernel(x, edge_index, edge_attr, enc_w, enc_b, dec_w, dec_b,
           msg_w1, msg_b1, msg_w2, msg_b2, upd_w1, upd_b1, upd_w2, upd_b2):
    l = msg_w1.shape[0] - 1
    W = msg_w1[l]
    wd, ws, we = W[:H], W[H:2 * H], W[2 * H:]

    h, a, b = _t1(x, enc_w, enc_b.reshape(1, H), wd, ws,
                  msg_b1[l].reshape(1, H))

    dst = edge_index[1]
    src = edge_index[0]
    ea0 = edge_attr[:, 0]
    ea1 = edge_attr[:, 1]
    ea2 = edge_attr[:, 2]
    z128 = jnp.zeros((ZR, H), _f32)
    z16 = jnp.zeros((ZR, 16), _f32)
    o16 = jnp.ones((C, 16), _f32)

    s_part, deg_part = _sc_call(a, b, dst, src, ea0, ea1, ea2, we,
                                z128, z16, o16)

    u1 = upd_w1[l]
    return _t2(h, s_part, deg_part,
               msg_w2[l], msg_b2[l].reshape(1, H),
               u1[:H], u1[H:], upd_b1[l].reshape(1, H),
               upd_w2[l], upd_b2[l].reshape(1, H),
               dec_w, dec_b.reshape(1, 5))


# speedup vs baseline: 1.8498x; 1.0606x over previous
import functools

import jax
import jax.numpy as jnp
from jax import lax
from jax.experimental import pallas as pl
from jax.experimental.pallas import tpu as pltpu
from jax.experimental.pallas import tpu_sc as plsc

N = 10000
E = 320000
H = 128

NC = 2
NS = 16
NW = NC * NS
EPW = E // NW
C = 40
NPH = 3
NSL = 4
NCHUNK = EPW // C
RPT = N // NS
ZR = 80

_f32 = jnp.float32


def _t1_body(x_ref, encw_ref, encb_ref, wd_ref, ws_ref, b1_ref,
             h_ref, a_ref, b_ref):
    h = jnp.dot(x_ref[...], encw_ref[...],
                preferred_element_type=_f32) + encb_ref[...]
    h_ref[...] = h
    a_ref[...] = jnp.dot(h, wd_ref[...],
                         preferred_element_type=_f32) + b1_ref[...]
    b_ref[...] = jnp.dot(h, ws_ref[...],
                         preferred_element_type=_f32)


def _t1(x, enc_w, enc_b, wd, ws, b1):
    blk = 1000
    grid = N // blk
    return pl.pallas_call(
        _t1_body,
        grid=(grid,),
        in_specs=[
            pl.BlockSpec((blk, 5), lambda i: (i, 0)),
            pl.BlockSpec((5, H), lambda i: (0, 0)),
            pl.BlockSpec((1, H), lambda i: (0, 0)),
            pl.BlockSpec((H, H), lambda i: (0, 0)),
            pl.BlockSpec((H, H), lambda i: (0, 0)),
            pl.BlockSpec((1, H), lambda i: (0, 0)),
        ],
        out_specs=[
            pl.BlockSpec((blk, H), lambda i: (i, 0)),
            pl.BlockSpec((blk, H), lambda i: (i, 0)),
            pl.BlockSpec((blk, H), lambda i: (i, 0)),
        ],
        out_shape=[jax.ShapeDtypeStruct((N, H), _f32)] * 3,
    )(x, enc_w, enc_b, wd, ws, b1)


def _sc_body(a_h, b_h, ei_h, ea_h, we_h,
             z128_h, z16_h, o16_h,
             s_out, deg_out,
             idxq, eaq, wev, arq, brq, onesv,
             s_sh, deg_sh, sem_i, sem_g, sem_s):
    cid = lax.axis_index("c")
    sid = lax.axis_index("s")
    wid = sid * NC + cid
    r0 = sid * RPT

    nfull, rem = RPT // ZR, RPT % ZR
    for q in range(nfull):
        pltpu.sync_copy(z128_h, s_sh.at[pl.ds(r0 + q * ZR, ZR)])
        pltpu.sync_copy(z16_h, deg_sh.at[pl.ds(r0 + q * ZR, ZR)])
    if rem:
        pltpu.sync_copy(z128_h.at[pl.ds(0, rem)],
                        s_sh.at[pl.ds(r0 + nfull * ZR, rem)])
        pltpu.sync_copy(z16_h.at[pl.ds(0, rem)],
                        deg_sh.at[pl.ds(r0 + nfull * ZR, rem)])

    pltpu.sync_copy(we_h, wev)
    pltpu.sync_copy(o16_h, onesv)
    plsc.subcore_barrier()

    wvecs = [[wev[r, pl.ds(k * 16, 16)] for k in range(8)] for r in range(3)]

    def idx_start(j, s):
        base = (wid * NCHUNK + j) * C
        pltpu.async_copy(ei_h.at[:, pl.ds(base, C)], idxq.at[s],
                         sem_i.at[s])
        pltpu.async_copy(ea_h.at[:, pl.ds(base, C)], eaq.at[s],
                         sem_i.at[s])

    def idx_wait(j, s):
        base = (wid * NCHUNK + j) * C
        pltpu.make_async_copy(ei_h.at[:, pl.ds(base, C)],
                              idxq.at[s], sem_i.at[s]).wait()
        pltpu.make_async_copy(ea_h.at[:, pl.ds(base, C)],
                              eaq.at[s], sem_i.at[s]).wait()

    def gather_start(s, p):
        pltpu.async_copy(a_h.at[idxq.at[s, 1]], arq.at[p], sem_g.at[p])
        pltpu.async_copy(b_h.at[idxq.at[s, 0]], brq.at[p], sem_g.at[p])

    def gather_wait(s, p):
        pltpu.make_async_copy(a_h.at[idxq.at[s, 1]], arq.at[p],
                              sem_g.at[p]).wait()
        pltpu.make_async_copy(b_h.at[idxq.at[s, 0]], brq.at[p],
                              sem_g.at[p]).wait()

    def scatter_start(s, p):
        pltpu.async_copy(arq.at[p], s_sh.at[idxq.at[s, 1]], sem_s.at[p],
                         add=True)
        pltpu.async_copy(onesv, deg_sh.at[idxq.at[s, 1]], sem_s.at[p],
                         add=True)

    def scatter_wait(s, p):
        pltpu.make_async_copy(arq.at[p], s_sh.at[idxq.at[s, 1]],
                              sem_s.at[p]).wait()
        pltpu.make_async_copy(onesv, deg_sh.at[idxq.at[s, 1]],
                              sem_s.at[p]).wait()

    def compute(s, p):
        @plsc.parallel_loop(0, C, unroll=4)
        def edge(e):
            gi = jnp.full((16,), e, jnp.int32)
            si = jnp.full((16,), s, jnp.int32)
            c0 = plsc.load_gather(eaq, [si, jnp.full((16,), 0, jnp.int32), gi])
            c1 = plsc.load_gather(eaq, [si, jnp.full((16,), 1, jnp.int32), gi])
            c2 = plsc.load_gather(eaq, [si, jnp.full((16,), 2, jnp.int32), gi])
            for k in range(8):
                sl = pl.ds(k * 16, 16)
                z = arq[p, e, sl] + brq[p, e, sl]
                z = z + c0 * wvecs[0][k] + c1 * wvecs[1][k] + c2 * wvecs[2][k]
                arq[p, e, sl] = jnp.maximum(z, 0.0)

    def chunk(j, carry):
        p = lax.rem(j, NPH)
        s = lax.rem(j, NSL)

        @pl.when(j >= 2)
        def _():
            scatter_wait(lax.rem(j - 2, NSL), lax.rem(j - 2, NPH))

        @pl.when(j + 2 < NCHUNK)
        def _():
            idx_start(j + 2, lax.rem(j + 2, NSL))

        @pl.when(j + 1 < NCHUNK)
        def _():
            s1 = lax.rem(j + 1, NSL)
            idx_wait(j + 1, s1)
            gather_start(s1, lax.rem(j + 1, NPH))

        gather_wait(s, p)
        compute(s, p)
        scatter_start(s, p)
        return carry

    idx_start(0, 0)
    idx_start(1, 1)
    idx_wait(0, 0)
    gather_start(0, 0)
    lax.fori_loop(0, NCHUNK, chunk, 0)
    scatter_wait((NCHUNK - 2) % NSL, (NCHUNK - 2) % NPH)
    scatter_wait((NCHUNK - 1) % NSL, (NCHUNK - 1) % NPH)
    plsc.subcore_barrier()

    pltpu.sync_copy(s_sh.at[pl.ds(r0, RPT)],
                    s_out.at[cid, pl.ds(r0, RPT)])
    pltpu.sync_copy(deg_sh.at[pl.ds(r0, RPT)],
                    deg_out.at[cid, pl.ds(r0, RPT)])


def _sc_call(a, b, ei, eaT, we, z128, z16, o16):
    mesh = plsc.VectorSubcoreMesh(core_axis_name="c", subcore_axis_name="s")
    return pl.kernel(
        _sc_body,
        out_type=[
            jax.ShapeDtypeStruct((NC, N, H), _f32),
            jax.ShapeDtypeStruct((NC, N, 16), _f32),
        ],
        mesh=mesh,
        compiler_params=pltpu.CompilerParams(use_tc_tiling_on_sc=False,
                                             needs_layout_passes=False),
        scratch_types=[
            pltpu.VMEM((NSL, 2, C), jnp.int32),
            pltpu.VMEM((NSL, 3, C), _f32),
            pltpu.VMEM((3, H), _f32),
            pltpu.VMEM((NPH, C, H), _f32),
            pltpu.VMEM((NPH, C, H), _f32),
            pltpu.VMEM((C, 16), _f32),
            pltpu.VMEM_SHARED((N, H), _f32),
            pltpu.VMEM_SHARED((N, 16), _f32),
            pltpu.SemaphoreType.DMA((NSL,)),
            pltpu.SemaphoreType.DMA((NPH,)),
            pltpu.SemaphoreType.DMA((NPH,)),
        ],
    )(a, b, ei, eaT, we, z128, z16, o16)


def _t2_body(h_ref, sp_ref, dp_ref, w2_ref, b2_ref,
             u1h_ref, u1a_ref, ub1_ref, u2_ref, ub2_ref, dw_ref, db_ref,
             o_ref):
    h = h_ref[...]
    s = sp_ref[0] + sp_ref[1]
    deg = dp_ref[0, :, :1] + dp_ref[1, :, :1]
    aggr = jnp.dot(s, w2_ref[...],
                   preferred_element_type=_f32) + deg * b2_ref[...]
    u = jnp.dot(h, u1h_ref[...], preferred_element_type=_f32)
    u = u + jnp.dot(aggr, u1a_ref[...],
                    preferred_element_type=_f32) + ub1_ref[...]
    u = jnp.maximum(u, 0.0)
    hu = jnp.dot(u, u2_ref[...], preferred_element_type=_f32) + ub2_ref[...]
    o_ref[...] = jnp.dot(h + hu, dw_ref[...],
                         preferred_element_type=_f32) + db_ref[...]


def _t2(h, s_part, deg_part, w2, b2, u1h, u1a, ub1, u2, ub2, dw, db):
    blk = 1000
    grid = N // blk
    full = lambda r, c: pl.BlockSpec((r, c), lambda i: (0, 0))
    row = lambda c: pl.BlockSpec((blk, c), lambda i: (i, 0))
    return pl.pallas_call(
        _t2_body,
        grid=(grid,),
        in_specs=[
            row(H),
            pl.BlockSpec((2, blk, H), lambda i: (0, i, 0)),
            pl.BlockSpec((2, blk, 16), lambda i: (0, i, 0)),
            full(H, H), full(1, H),
            full(H, H), full(H, H), full(1, H),
            full(H, H), full(1, H),
            full(H, 5), full(1, 5),
        ],
        out_specs=pl.BlockSpec((blk, 5), lambda i: (i, 0)),
        out_shape=jax.ShapeDtypeStruct((N, 5), _f32),
    )(h, s_part, deg_part, w2, b2, u1h, u1a, ub1, u2, ub2, dw, db)


def kernel(x, edge_index, edge_attr, enc_w, enc_b, dec_w, dec_b,
           msg_w1, msg_b1, msg_w2, msg_b2, upd_w1, upd_b1, upd_w2, upd_b2):
    l = msg_w1.shape[0] - 1
    W = msg_w1[l]
    wd, ws, we = W[:H], W[H:2 * H], W[2 * H:]

    h, a, b = _t1(x, enc_w, enc_b.reshape(1, H), wd, ws,
                  msg_b1[l].reshape(1, H))

    eaT = edge_attr.T
    z128 = jnp.zeros((ZR, H), _f32)
    z16 = jnp.zeros((ZR, 16), _f32)
    o16 = jnp.ones((C, 16), _f32)

    s_part, deg_part = _sc_call(a, b, edge_index, eaT, we, z128, z16, o16)

    u1 = upd_w1[l]
    return _t2(h, s_part, deg_part,
               msg_w2[l], msg_b2[l].reshape(1, H),
               u1[:H], u1[H:], upd_b1[l].reshape(1, H),
               upd_w2[l], upd_b2[l].reshape(1, H),
               dec_w, dec_b.reshape(1, 5))


# speedup vs baseline: 1.8886x; 1.0210x over previous
import functools

import jax
import jax.numpy as jnp
from jax import lax
from jax.experimental import pallas as pl
from jax.experimental.pallas import tpu as pltpu
from jax.experimental.pallas import tpu_sc as plsc

N = 10000
E = 320000
H = 128

NC = 2
NS = 16
NW = NC * NS
EPW = E // NW
C = 40
NPH = 3
NSL = 4
NCHUNK = EPW // C
RPT = N // NS
ZR = 80

_f32 = jnp.float32


def _t1_body(x_ref, encw_ref, encb_ref, wd_ref, ws_ref, b1_ref,
             h_ref, a_ref, b_ref):
    h = jnp.dot(x_ref[...], encw_ref[...],
                preferred_element_type=_f32) + encb_ref[...]
    h_ref[...] = h
    a_ref[...] = jnp.dot(h, wd_ref[...],
                         preferred_element_type=_f32) + b1_ref[...]
    b_ref[...] = jnp.dot(h, ws_ref[...],
                         preferred_element_type=_f32)


def _t1(x, enc_w, enc_b, wd, ws, b1):
    blk = 1000
    grid = N // blk
    return pl.pallas_call(
        _t1_body,
        grid=(grid,),
        in_specs=[
            pl.BlockSpec((blk, 5), lambda i: (i, 0)),
            pl.BlockSpec((5, H), lambda i: (0, 0)),
            pl.BlockSpec((1, H), lambda i: (0, 0)),
            pl.BlockSpec((H, H), lambda i: (0, 0)),
            pl.BlockSpec((H, H), lambda i: (0, 0)),
            pl.BlockSpec((1, H), lambda i: (0, 0)),
        ],
        out_specs=[
            pl.BlockSpec((blk, H), lambda i: (i, 0)),
            pl.BlockSpec((blk, H), lambda i: (i, 0)),
            pl.BlockSpec((blk, H), lambda i: (i, 0)),
        ],
        out_shape=[jax.ShapeDtypeStruct((N, H), _f32)] * 3,
    )(x, enc_w, enc_b, wd, ws, b1)


def _sc_body(a_h, b_h, ei_h, ea_h, we_h,
             z128_h, z16_h, o16_h,
             s_out, deg_out,
             idxq, eaq, wev, arq, brq, onesv,
             s_sh, deg_sh, sem_i, sem_g, sem_s):
    cid = lax.axis_index("c")
    sid = lax.axis_index("s")
    wid = sid * NC + cid
    r0 = sid * RPT

    nfull, rem = RPT // ZR, RPT % ZR
    for q in range(nfull):
        pltpu.sync_copy(z128_h, s_sh.at[pl.ds(r0 + q * ZR, ZR)])
        pltpu.sync_copy(z16_h, deg_sh.at[pl.ds(r0 + q * ZR, ZR)])
    if rem:
        pltpu.sync_copy(z128_h.at[pl.ds(0, rem)],
                        s_sh.at[pl.ds(r0 + nfull * ZR, rem)])
        pltpu.sync_copy(z16_h.at[pl.ds(0, rem)],
                        deg_sh.at[pl.ds(r0 + nfull * ZR, rem)])

    pltpu.sync_copy(we_h, wev)
    pltpu.sync_copy(o16_h, onesv)
    plsc.subcore_barrier()

    wvecs = [[wev[r, pl.ds(k * 16, 16)] for k in range(8)] for r in range(3)]

    def idx_start(j, s):
        base = (wid * NCHUNK + j) * C
        pltpu.async_copy(ei_h.at[:, pl.ds(base, C)], idxq.at[s],
                         sem_i.at[s])
        pltpu.async_copy(ea_h.at[:, pl.ds(base, C)], eaq.at[s],
                         sem_i.at[s])

    def idx_wait(j, s):
        base = (wid * NCHUNK + j) * C
        pltpu.make_async_copy(ei_h.at[:, pl.ds(base, C)],
                              idxq.at[s], sem_i.at[s]).wait()
        pltpu.make_async_copy(ea_h.at[:, pl.ds(base, C)],
                              eaq.at[s], sem_i.at[s]).wait()

    def gather_start(s, p):
        pltpu.async_copy(a_h.at[idxq.at[s, 1]], arq.at[p], sem_g.at[p])
        pltpu.async_copy(b_h.at[idxq.at[s, 0]], brq.at[p], sem_g.at[p])

    def gather_wait(s, p):
        pltpu.make_async_copy(a_h.at[idxq.at[s, 1]], arq.at[p],
                              sem_g.at[p]).wait()
        pltpu.make_async_copy(b_h.at[idxq.at[s, 0]], brq.at[p],
                              sem_g.at[p]).wait()

    def scatter_start(s, p):
        pltpu.async_copy(arq.at[p], s_sh.at[idxq.at[s, 1]], sem_s.at[p],
                         add=True)
        pltpu.async_copy(onesv, deg_sh.at[idxq.at[s, 1]], sem_s.at[p],
                         add=True)

    def scatter_wait(s, p):
        pltpu.make_async_copy(arq.at[p], s_sh.at[idxq.at[s, 1]],
                              sem_s.at[p]).wait()
        pltpu.make_async_copy(onesv, deg_sh.at[idxq.at[s, 1]],
                              sem_s.at[p]).wait()

    def compute(s, p):
        @plsc.parallel_loop(0, C, unroll=8)
        def edge(e):
            gi = jnp.full((16,), e, jnp.int32)
            si = jnp.full((16,), s, jnp.int32)
            c0 = plsc.load_gather(eaq, [si, jnp.full((16,), 0, jnp.int32), gi])
            c1 = plsc.load_gather(eaq, [si, jnp.full((16,), 1, jnp.int32), gi])
            c2 = plsc.load_gather(eaq, [si, jnp.full((16,), 2, jnp.int32), gi])
            for k in range(8):
                sl = pl.ds(k * 16, 16)
                z = arq[p, e, sl] + brq[p, e, sl]
                z = z + c0 * wvecs[0][k] + c1 * wvecs[1][k] + c2 * wvecs[2][k]
                arq[p, e, sl] = jnp.maximum(z, 0.0)

    def chunk(j, carry):
        p = lax.rem(j, NPH)
        s = lax.rem(j, NSL)

        @pl.when(j >= 2)
        def _():
            scatter_wait(lax.rem(j - 2, NSL), lax.rem(j - 2, NPH))

        @pl.when(j + 2 < NCHUNK)
        def _():
            idx_start(j + 2, lax.rem(j + 2, NSL))

        @pl.when(j + 1 < NCHUNK)
        def _():
            s1 = lax.rem(j + 1, NSL)
            idx_wait(j + 1, s1)
            gather_start(s1, lax.rem(j + 1, NPH))

        gather_wait(s, p)
        compute(s, p)
        scatter_start(s, p)
        return carry

    idx_start(0, 0)
    idx_start(1, 1)
    idx_wait(0, 0)
    gather_start(0, 0)
    lax.fori_loop(0, NCHUNK, chunk, 0)
    scatter_wait((NCHUNK - 2) % NSL, (NCHUNK - 2) % NPH)
    scatter_wait((NCHUNK - 1) % NSL, (NCHUNK - 1) % NPH)
    plsc.subcore_barrier()

    pltpu.sync_copy(s_sh.at[pl.ds(r0, RPT)],
                    s_out.at[cid, pl.ds(r0, RPT)])
    pltpu.sync_copy(deg_sh.at[pl.ds(r0, RPT)],
                    deg_out.at[cid, pl.ds(r0, RPT)])


def _sc_call(a, b, ei, eaT, we, z128, z16, o16):
    mesh = plsc.VectorSubcoreMesh(core_axis_name="c", subcore_axis_name="s")
    return pl.kernel(
        _sc_body,
        out_type=[
            jax.ShapeDtypeStruct((NC, N, H), _f32),
            jax.ShapeDtypeStruct((NC, N, 16), _f32),
        ],
        mesh=mesh,
        compiler_params=pltpu.CompilerParams(use_tc_tiling_on_sc=False,
                                             needs_layout_passes=False),
        scratch_types=[
            pltpu.VMEM((NSL, 2, C), jnp.int32),
            pltpu.VMEM((NSL, 3, C), _f32),
            pltpu.VMEM((3, H), _f32),
            pltpu.VMEM((NPH, C, H), _f32),
            pltpu.VMEM((NPH, C, H), _f32),
            pltpu.VMEM((C, 16), _f32),
            pltpu.VMEM_SHARED((N, H), _f32),
            pltpu.VMEM_SHARED((N, 16), _f32),
            pltpu.SemaphoreType.DMA((NSL,)),
            pltpu.SemaphoreType.DMA((NPH,)),
            pltpu.SemaphoreType.DMA((NPH,)),
        ],
    )(a, b, ei, eaT, we, z128, z16, o16)


def _t2_body(h_ref, sp_ref, dp_ref, w2_ref, b2_ref,
             u1h_ref, u1a_ref, ub1_ref, u2_ref, ub2_ref, dw_ref, db_ref,
             o_ref):
    h = h_ref[...]
    s = sp_ref[0] + sp_ref[1]
    deg = dp_ref[0, :, :1] + dp_ref[1, :, :1]
    aggr = jnp.dot(s, w2_ref[...],
                   preferred_element_type=_f32) + deg * b2_ref[...]
    u = jnp.dot(h, u1h_ref[...], preferred_element_type=_f32)
    u = u + jnp.dot(aggr, u1a_ref[...],
                    preferred_element_type=_f32) + ub1_ref[...]
    u = jnp.maximum(u, 0.0)
    hu = jnp.dot(u, u2_ref[...], preferred_element_type=_f32) + ub2_ref[...]
    o_ref[...] = jnp.dot(h + hu, dw_ref[...],
                         preferred_element_type=_f32) + db_ref[...]


def _t2(h, s_part, deg_part, w2, b2, u1h, u1a, ub1, u2, ub2, dw, db):
    blk = 1000
    grid = N // blk
    full = lambda r, c: pl.BlockSpec((r, c), lambda i: (0, 0))
    row = lambda c: pl.BlockSpec((blk, c), lambda i: (i, 0))
    return pl.pallas_call(
        _t2_body,
        grid=(grid,),
        in_specs=[
            row(H),
            pl.BlockSpec((2, blk, H), lambda i: (0, i, 0)),
            pl.BlockSpec((2, blk, 16), lambda i: (0, i, 0)),
            full(H, H), full(1, H),
            full(H, H), full(H, H), full(1, H),
            full(H, H), full(1, H),
            full(H, 5), full(1, 5),
        ],
        out_specs=pl.BlockSpec((blk, 5), lambda i: (i, 0)),
        out_shape=jax.ShapeDtypeStruct((N, 5), _f32),
    )(h, s_part, deg_part, w2, b2, u1h, u1a, ub1, u2, ub2, dw, db)


def kernel(x, edge_index, edge_attr, enc_w, enc_b, dec_w, dec_b,
           msg_w1, msg_b1, msg_w2, msg_b2, upd_w1, upd_b1, upd_w2, upd_b2):
    l = msg_w1.shape[0] - 1
    W = msg_w1[l]
    wd, ws, we = W[:H], W[H:2 * H], W[2 * H:]

    h, a, b = _t1(x, enc_w, enc_b.reshape(1, H), wd, ws,
                  msg_b1[l].reshape(1, H))

    eaT = edge_attr.T
    z128 = jnp.zeros((ZR, H), _f32)
    z16 = jnp.zeros((ZR, 16), _f32)
    o16 = jnp.ones((C, 16), _f32)

    s_part, deg_part = _sc_call(a, b, edge_index, eaT, we, z128, z16, o16)

    u1 = upd_w1[l]
    return _t2(h, s_part, deg_part,
               msg_w2[l], msg_b2[l].reshape(1, H),
               u1[:H], u1[H:], upd_b1[l].reshape(1, H),
               upd_w2[l], upd_b2[l].reshape(1, H),
               dec_w, dec_b.reshape(1, 5))
